# baseline (speedup 1.0000x reference)
import jax
import jax.numpy as jnp
from jax import lax
from jax.experimental import pallas as pl
from jax.experimental.pallas import tpu as pltpu
from jax.experimental.pallas import tpu_sc as plsc

BATCH = 4
SEQ = 2048
D = 64
NTOK = BATCH * SEQ

NC = 2
NS = 16
L = 16
NW = NC * NS
B_W = NTOK // NW
CHUNK = 128
NCH = B_W // CHUNK


def _body(x_hbm, tok_hbm, pos_hbm, out_hbm, idx_v, acc_v, pos_v, sem):
    wid = lax.axis_index("s") * NC + lax.axis_index("c")
    base = wid * B_W
    pltpu.sync_copy(x_hbm.at[pl.ds(wid * NCH, NCH)], idx_v)
    copies = [
        pltpu.async_copy(
            tok_hbm.at[idx_v.at[c]], acc_v.at[pl.ds(c * CHUNK, CHUNK)], sem
        )
        for c in range(NCH)
    ]
    pos_base = lax.rem(base, SEQ)
    pltpu.sync_copy(pos_hbm.at[pl.ds(pos_base, B_W)], pos_v)
    for cp in copies:
        cp.wait()

    def _row(r, carry):
        for c in range(D // L):
            sl = pl.ds(c * L, L)
            acc_v[r, sl] += pos_v[r, sl]
        return carry

    lax.fori_loop(0, B_W, _row, 0)
    pltpu.sync_copy(acc_v, out_hbm.at[pl.ds(base, B_W)])


def kernel(x, token_table, pos_table):
    x2 = x.reshape(NW * NCH, CHUNK)
    mesh = plsc.VectorSubcoreMesh(core_axis_name="c", subcore_axis_name="s")
    out = pl.kernel(
        _body,
        mesh=mesh,
        out_type=jax.ShapeDtypeStruct((NTOK, D), jnp.float32),
        scratch_types=[
            pltpu.VMEM((NCH, CHUNK), jnp.int32),
            pltpu.VMEM((B_W, D), jnp.float32),
            pltpu.VMEM((B_W, D), jnp.float32),
            pltpu.SemaphoreType.DMA,
        ],
        compiler_params=pltpu.CompilerParams(use_tc_tiling_on_sc=False),
    )(x2, token_table, pos_table)
    return out.reshape(BATCH, SEQ, D)
